# baseline (speedup 1.0000x reference)
import jax
import jax.numpy as jnp
from jax import lax
from jax.experimental import pallas as pl
from jax.experimental.pallas import tpu as pltpu
from jax.experimental.pallas import tpu_sc as plsc

DEP_VOCAB = 50
EMBED_DIM = 64
B, S = 8, 256
N = B * S * S
NC, NS = 2, 16
NW = NC * NS
NROWS = B * S
ROWS_PER_W = NROWS // NW
IN_ROWS = 8
OUT_ROWS = 2
SUBS = IN_ROWS // OUT_ROWS
NIN = ROWS_PER_W // IN_ROWS
NOUT = ROWS_PER_W // OUT_ROWS
LANES = 16


def _sc_body(idx_hbm, adj_hbm, tab_hbm, out_hbm, tab_v, idx_vs, adj_vs,
             out_vs, in_sems, out_sems):
    wid = lax.axis_index("s") * NC + lax.axis_index("c")
    row0 = wid * ROWS_PER_W
    pltpu.sync_copy(tab_hbm, tab_v)

    def in_descs(k, buf):
        rbase = jnp.minimum(row0 + k * IN_ROWS, NROWS - IN_ROWS)
        bb = rbase >> 8
        ii = pl.multiple_of(rbase & (S - 1), IN_ROWS)
        return (
            pltpu.make_async_copy(idx_hbm.at[bb, pl.ds(ii, IN_ROWS)],
                                  idx_vs[buf], in_sems[buf]),
            pltpu.make_async_copy(adj_hbm.at[bb, pl.ds(ii, IN_ROWS)],
                                  adj_vs[buf], in_sems[buf]),
        )

    def out_desc(ci, obuf):
        rbase = row0 + ci * OUT_ROWS
        return pltpu.make_async_copy(
            out_vs[obuf],
            out_hbm.at[pl.ds(rbase * EMBED_DIM, OUT_ROWS * EMBED_DIM)],
            out_sems[obuf])

    def compute(buf, sub, obuf):
        ivb = idx_vs[buf]
        avb = adj_vs[buf]
        ovb = out_vs[obuf]

        def group_body(g, c2):
            r0 = g >> 4
            j0 = (g & 15) << 4
            idx16 = ivb[sub * OUT_ROWS + r0, pl.ds(j0, LANES)]
            adj16 = avb[sub * OUT_ROWS + r0, pl.ds(j0, LANES)]
            idx64 = idx16 * EMBED_DIM
            iota = lax.iota(jnp.int32, LANES)
            jvec = iota + j0
            rb64 = r0 * EMBED_DIM

            @plsc.parallel_loop(0, LANES, unroll=8)
            def s_loop(s):
                ps = (iota + s) & (LANES - 1)
                idxp = idx64 + ps
                rd = ps + rb64
                for c in range(4):
                    r = plsc.load_gather(tab_v, [idxp + c * LANES])
                    plsc.store_scatter(ovb, [rd + c * LANES, jvec],
                                       r * adj16)

            return c2

        lax.fori_loop(0, OUT_ROWS * S // LANES, group_body, 0)

    for d in in_descs(0, 0) + in_descs(1, 1):
        d.start()

    def super_body(sc, carry):
        for buf in range(2):
            k = sc * 2 + buf
            for d in in_descs(k, buf):
                d.wait()
            for sub in range(SUBS):
                ci = k * SUBS + sub
                obuf = sub & 1

                if buf * SUBS + sub >= 2:
                    out_desc(ci - 2, obuf).wait()
                else:
                    @pl.when(sc >= 1)
                    def _():
                        out_desc(ci - 2, obuf).wait()

                compute(buf, sub, obuf)
                out_desc(ci, obuf).start()
            for d in in_descs(k + 2, buf):
                d.start()
        return carry

    lax.fori_loop(0, NIN // 2, super_body, 0)
    out_desc(NOUT - 2, 0).wait()
    out_desc(NOUT - 1, 1).wait()
    for buf in range(2):
        for d in in_descs(NIN + buf, buf):
            d.wait()


@jax.jit
def _sc_call(idx, adjf, tab):
    mesh = plsc.VectorSubcoreMesh(core_axis_name="c", subcore_axis_name="s",
                                  num_cores=NC, num_subcores=NS)
    fn = pl.kernel(
        _sc_body,
        out_type=jax.ShapeDtypeStruct((NROWS * EMBED_DIM, S), jnp.float32),
        mesh=mesh,
        compiler_params=pltpu.CompilerParams(needs_layout_passes=False),
        scratch_types=[
            pltpu.VMEM((DEP_VOCAB * EMBED_DIM,), jnp.float32),
            [pltpu.VMEM((IN_ROWS, S), jnp.int32),
             pltpu.VMEM((IN_ROWS, S), jnp.int32)],
            [pltpu.VMEM((IN_ROWS, S), jnp.float32),
             pltpu.VMEM((IN_ROWS, S), jnp.float32)],
            [pltpu.VMEM((OUT_ROWS * EMBED_DIM, S), jnp.float32),
             pltpu.VMEM((OUT_ROWS * EMBED_DIM, S), jnp.float32)],
            [pltpu.SemaphoreType.DMA, pltpu.SemaphoreType.DMA],
            [pltpu.SemaphoreType.DMA, pltpu.SemaphoreType.DMA],
        ],
    )
    return fn(idx, adjf, tab)


def kernel(adj_matrix, dep_rel_matrix, dep_table):
    idx = dep_rel_matrix.astype(jnp.int32)
    adjf = adj_matrix.astype(jnp.float32)
    tab = dep_table.reshape(-1).astype(jnp.float32)
    out = _sc_call(idx, adjf, tab)
    return out.reshape(B, S, EMBED_DIM, S).transpose(0, 1, 3, 2)


# speedup vs baseline: 1.0764x; 1.0764x over previous
import jax
import jax.numpy as jnp
from jax import lax
from jax.experimental import pallas as pl
from jax.experimental.pallas import tpu as pltpu
from jax.experimental.pallas import tpu_sc as plsc

DEP_VOCAB = 50
EMBED_DIM = 64
B, S = 8, 256
N = B * S * S
NC, NS = 2, 16
NW = NC * NS
NROWS = B * S
ROWS_PER_W = NROWS // NW
IN_ROWS = 8
OUT_ROWS = 2
SUBS = IN_ROWS // OUT_ROWS
NIN = ROWS_PER_W // IN_ROWS
NOUT = ROWS_PER_W // OUT_ROWS
LANES = 16


def _sc_body(idx_hbm, adj_hbm, tab_hbm, out_hbm, tab_v, idx_vs, adj_vs,
             out_vs, in_sems, out_sems):
    wid = lax.axis_index("s") * NC + lax.axis_index("c")
    row0 = wid * ROWS_PER_W
    pltpu.sync_copy(tab_hbm, tab_v)

    def in_descs(k, buf):
        rbase = jnp.minimum(row0 + k * IN_ROWS, NROWS - IN_ROWS)
        bb = rbase >> 8
        ii = pl.multiple_of(rbase & (S - 1), IN_ROWS)
        return (
            pltpu.make_async_copy(idx_hbm.at[bb, pl.ds(ii, IN_ROWS)],
                                  idx_vs[buf], in_sems[buf]),
            pltpu.make_async_copy(adj_hbm.at[bb, pl.ds(ii, IN_ROWS)],
                                  adj_vs[buf], in_sems[buf]),
        )

    def out_desc(ci, obuf):
        rbase = row0 + ci * OUT_ROWS
        return pltpu.make_async_copy(
            out_vs[obuf],
            out_hbm.at[pl.ds(rbase * EMBED_DIM, OUT_ROWS * EMBED_DIM)],
            out_sems[obuf])

    def compute(buf, sub, obuf):
        ivb = idx_vs[buf]
        avb = adj_vs[buf]
        ovb = out_vs[obuf]

        def group_body(g, c2):
            r0 = g >> 4
            j0 = (g & 15) << 4
            idx16 = ivb[sub * OUT_ROWS + r0, pl.ds(j0, LANES)]
            adj16 = avb[sub * OUT_ROWS + r0, pl.ds(j0, LANES)]
            idx64 = idx16 * EMBED_DIM
            iota = lax.iota(jnp.int32, LANES)
            jvec = iota + j0
            rb64 = r0 * EMBED_DIM

            @plsc.parallel_loop(0, LANES, unroll=4)
            def s_loop(s):
                ps = (iota + s) & (LANES - 1)
                idxp = idx64 + ps
                rd = ps + rb64
                for c in range(4):
                    r = plsc.load_gather(tab_v, [idxp + c * LANES])
                    plsc.store_scatter(ovb, [rd + c * LANES, jvec],
                                       r * adj16)

            return c2

        lax.fori_loop(0, OUT_ROWS * S // LANES, group_body, 0)

    for d in in_descs(0, 0) + in_descs(1, 1):
        d.start()

    def super_body(sc, carry):
        for buf in range(2):
            k = sc * 2 + buf
            for d in in_descs(k, buf):
                d.wait()
            for sub in range(SUBS):
                ci = k * SUBS + sub
                obuf = sub & 1

                if buf * SUBS + sub >= 2:
                    out_desc(ci - 2, obuf).wait()
                else:
                    @pl.when(sc >= 1)
                    def _():
                        out_desc(ci - 2, obuf).wait()

                compute(buf, sub, obuf)
                out_desc(ci, obuf).start()
            for d in in_descs(k + 2, buf):
                d.start()
        return carry

    lax.fori_loop(0, NIN // 2, super_body, 0)
    out_desc(NOUT - 2, 0).wait()
    out_desc(NOUT - 1, 1).wait()
    for buf in range(2):
        for d in in_descs(NIN + buf, buf):
            d.wait()


@jax.jit
def _sc_call(idx, adjf, tab):
    mesh = plsc.VectorSubcoreMesh(core_axis_name="c", subcore_axis_name="s",
                                  num_cores=NC, num_subcores=NS)
    fn = pl.kernel(
        _sc_body,
        out_type=jax.ShapeDtypeStruct((NROWS * EMBED_DIM, S), jnp.float32),
        mesh=mesh,
        compiler_params=pltpu.CompilerParams(needs_layout_passes=False),
        scratch_types=[
            pltpu.VMEM((DEP_VOCAB * EMBED_DIM,), jnp.float32),
            [pltpu.VMEM((IN_ROWS, S), jnp.int32),
             pltpu.VMEM((IN_ROWS, S), jnp.int32)],
            [pltpu.VMEM((IN_ROWS, S), jnp.float32),
             pltpu.VMEM((IN_ROWS, S), jnp.float32)],
            [pltpu.VMEM((OUT_ROWS * EMBED_DIM, S), jnp.float32),
             pltpu.VMEM((OUT_ROWS * EMBED_DIM, S), jnp.float32)],
            [pltpu.SemaphoreType.DMA, pltpu.SemaphoreType.DMA],
            [pltpu.SemaphoreType.DMA, pltpu.SemaphoreType.DMA],
        ],
    )
    return fn(idx, adjf, tab)


def kernel(adj_matrix, dep_rel_matrix, dep_table):
    idx = dep_rel_matrix.astype(jnp.int32)
    adjf = adj_matrix.astype(jnp.float32)
    tab = dep_table.reshape(-1).astype(jnp.float32)
    out = _sc_call(idx, adjf, tab)
    return out.reshape(B, S, EMBED_DIM, S).transpose(0, 1, 3, 2)
